# baseline (speedup 1.0000x reference)
import functools

import jax
import jax.numpy as jnp
from jax import lax
from jax.experimental import pallas as pl
from jax.experimental.pallas import tpu as pltpu
from jax.experimental.pallas import tpu_sc as plsc

_NUM_DOMAINS = 26
_VOCAB = 100000
_EMB = 16
_BATCH = 16384

_NUM_CORES = 2
_NUM_SUBCORES = 16
_NW = _NUM_CORES * _NUM_SUBCORES
_ROWS_W = _BATCH // _NW
_CHUNK = 128
_NCHUNK = _ROWS_W // _CHUNK
_UNROLL = 4


_sc_mesh = plsc.VectorSubcoreMesh(
    core_axis_name="c", subcore_axis_name="s",
    num_cores=_NUM_CORES, num_subcores=_NUM_SUBCORES)


@functools.partial(
    pl.kernel,
    out_type=jax.ShapeDtypeStruct((_BATCH, _EMB), jnp.float32),
    mesh=_sc_mesh,
    scratch_types=[
        pltpu.VMEM((_NUM_DOMAINS, _NCHUNK, _CHUNK), jnp.int32),
        pltpu.VMEM((_NUM_DOMAINS, _CHUNK, _EMB), jnp.float32),
        pltpu.VMEM((_CHUNK, _EMB), jnp.float32),
        pltpu.SemaphoreType.DMA,
    ],
    compiler_params=pltpu.CompilerParams(use_tc_tiling_on_sc=False),
)
def _sc_gather_sum(table_hbm, idx_hbm, out_hbm, idx_v, buf, acc, sem):
    wid = lax.axis_index("s") * _NUM_CORES + lax.axis_index("c")
    base = wid * _ROWS_W
    pltpu.sync_copy(idx_hbm.at[wid], idx_v)
    for c in range(_NCHUNK):
        copies = []
        for i in range(_NUM_DOMAINS):
            copies.append(
                pltpu.async_copy(table_hbm.at[idx_v.at[i, c]], buf.at[i], sem))
        for cp in copies:
            cp.wait()
        def body(r0, _):
            for u in range(_UNROLL):
                r = r0 * _UNROLL + u
                s = buf[0, r, :]
                for i in range(1, _NUM_DOMAINS):
                    s = s + buf[i, r, :]
                acc[r, :] = s
            return 0
        lax.fori_loop(0, _CHUNK // _UNROLL, body, 0)
        pltpu.sync_copy(acc, out_hbm.at[pl.ds(base + c * _CHUNK, _CHUNK)])


def _finish_body(s_ref, o_ref):
    s = s_ref[...]
    s2 = jnp.sum(s * s, axis=1)
    o_ref[...] = jnp.tanh(jnp.sqrt(s2))


def _finish(summed):
    return pl.pallas_call(
        _finish_body,
        out_shape=jax.ShapeDtypeStruct((_BATCH,), jnp.float32),
    )(summed)


def kernel(x, tables):
    flat_table = tables.reshape(_NUM_DOMAINS * _VOCAB, _EMB)
    offs = jnp.arange(_NUM_DOMAINS, dtype=jnp.int32) * _VOCAB
    idx = (x + offs[None, :]).T.reshape(_NUM_DOMAINS, _NW, _ROWS_W)
    idx = idx.transpose(1, 0, 2).reshape(_NW, _NUM_DOMAINS, _NCHUNK, _CHUNK)
    summed = _sc_gather_sum(flat_table, idx)
    return _finish(summed)
